# baseline (speedup 1.0000x reference)
import functools

import jax
import jax.numpy as jnp
from jax import lax
from jax.experimental import pallas as pl
from jax.experimental.pallas import tpu as pltpu
from jax.experimental.pallas import tpu_sc as plsc

N = 10000
E = 320000
F = 128
NCLASS = 64

NC = 2
NS = 16
NW = NC * NS
C = 48
EPW = 10368
EPAD = EPW * NW
NCHUNK = EPW // C
SCH = 36
NSUP = NCHUNK // SCH
NBUF = 4
ZR = 40
NPAD = 10240
ROWS_PT = NPAD // NS


def _i16(v):
    return jnp.full((16,), v, dtype=jnp.int32)


def _spmm_body(hn, srcr, dstr, wr, out, acc, src_v, dst_v, w_v,
               rows0, rows1, rows2, rows3, gsem0, gsem1, gsem2, gsem3,
               ssem0, ssem1, ssem2, ssem3):
    c = lax.axis_index("c")
    s = lax.axis_index("s")
    wid = s * NC + c

    rows = (rows0, rows1, rows2, rows3)
    gsems = (gsem0, gsem1, gsem2, gsem3)
    ssems = (ssem0, ssem1, ssem2, ssem3)

    def gather_start(k, b):
        pltpu.async_copy(hn.at[src_v.at[k]], rows[b], gsems[b])

    def gather_wait(b):
        pltpu.make_async_copy(hn.at[src_v.at[0]], rows[b], gsems[b]).wait()

    def scatter_start(k, b):
        pltpu.async_copy(rows[b], acc.at[dst_v.at[k]], ssems[b], add=True)

    def scatter_wait(b):
        pltpu.make_async_copy(rows[b], acc.at[dst_v.at[0]], ssems[b]).wait()

    def multiply(k, rows_b):
        def group(g, carry2):
            w16 = w_v[k, pl.ds(g * 16, 16)]
            for j in range(16):
                wvec = jnp.full((16,), w16[j], dtype=jnp.float32)
                e = g * 16 + j
                for f in range(F // 16):
                    sl = (e, pl.ds(f * 16, 16))
                    rows_b[sl] = rows_b[sl] * wvec
            return carry2

        lax.fori_loop(0, C // 16, group, 0, unroll=False)

    def zrow(i, carry):
        for f in range(F // 16):
            rows0[i, pl.ds(f * 16, 16)] = jnp.zeros((16,), jnp.float32)
        return carry

    lax.fori_loop(0, ZR, zrow, 0, unroll=False)
    for t in range(ROWS_PT // ZR):
        pltpu.sync_copy(rows0.at[pl.ds(0, ZR)],
                        acc.at[pl.ds(s * ROWS_PT + t * ZR, ZR)])
    plsc.subcore_barrier()

    def superchunk(sc_i, carry0):
        pltpu.sync_copy(srcr.at[wid, sc_i], src_v)
        pltpu.sync_copy(dstr.at[wid, sc_i], dst_v)
        pltpu.sync_copy(wr.at[wid, sc_i], w_v)
        gather_start(0, 0)
        gather_start(1, 1)

        def quad(q, carry):
            for b in range(NBUF):
                k = 4 * q + b
                b2 = (b + 2) % NBUF
                gather_wait(b)
                multiply(k, rows[b])
                scatter_start(k, b)

                @pl.when(k >= 2)
                def _():
                    scatter_wait(b2)

                @pl.when(k + 2 < SCH)
                def _():
                    gather_start(k + 2, b2)
            return carry

        lax.fori_loop(0, SCH // NBUF, quad, 0, unroll=False)
        scatter_wait((SCH - 2) % NBUF)
        scatter_wait((SCH - 1) % NBUF)
        return carry0

    lax.fori_loop(0, NSUP, superchunk, 0, unroll=False)
    plsc.subcore_barrier()
    pltpu.sync_copy(acc.at[pl.ds(s * ROWS_PT, ROWS_PT)],
                    out.at[c, pl.ds(s * ROWS_PT, ROWS_PT)])


_spmm = pl.kernel(
    _spmm_body,
    out_type=jax.ShapeDtypeStruct((NC, NPAD, F), jnp.float32),
    mesh=plsc.VectorSubcoreMesh(core_axis_name="c", subcore_axis_name="s"),
    scratch_types=[
        pltpu.VMEM_SHARED((NPAD, F), jnp.float32),
        pltpu.VMEM((SCH, C), jnp.int32),
        pltpu.VMEM((SCH, C), jnp.int32),
        pltpu.VMEM((SCH, C), jnp.float32),
        pltpu.VMEM((C, F), jnp.float32),
        pltpu.VMEM((C, F), jnp.float32),
        pltpu.VMEM((C, F), jnp.float32),
        pltpu.VMEM((C, F), jnp.float32),
        pltpu.SemaphoreType.DMA,
        pltpu.SemaphoreType.DMA,
        pltpu.SemaphoreType.DMA,
        pltpu.SemaphoreType.DMA,
        pltpu.SemaphoreType.DMA,
        pltpu.SemaphoreType.DMA,
        pltpu.SemaphoreType.DMA,
        pltpu.SemaphoreType.DMA,
    ],
)


def _layernorm(h, w, b):
    m = jnp.mean(h, axis=-1, keepdims=True)
    v = jnp.mean((h - m) * (h - m), axis=-1, keepdims=True)
    return (h - m) * lax.rsqrt(v + 1e-5) * w + b


def _tc_in_body(x_ref, w_ref, b_ref, lnw_ref, lnb_ref, h_ref, hn_ref):
    h = jnp.dot(x_ref[...], w_ref[...], preferred_element_type=jnp.float32)
    h = jnp.maximum(h + b_ref[...], 0.0)
    h_ref[...] = h
    hn_ref[...] = _layernorm(h, lnw_ref[...], lnb_ref[...])


def _tc_mid_body(s2_ref, h_ref, w_ref, b_ref, lnw_ref, lnb_ref, ho_ref, hn_ref):
    sp = s2_ref[0] + s2_ref[1]
    t = jnp.dot(sp, w_ref[...], preferred_element_type=jnp.float32)
    t = jnp.maximum(t + b_ref[...], 0.0)
    h = h_ref[...] + t
    ho_ref[...] = h
    hn_ref[...] = _layernorm(h, lnw_ref[...], lnb_ref[...])


def _tc_last_body(s2_ref, h_ref, w_ref, b_ref, lnw_ref, lnb_ref,
                  wout_ref, bout_ref, z_ref):
    sp = s2_ref[0] + s2_ref[1]
    t = jnp.dot(sp, w_ref[...], preferred_element_type=jnp.float32) + b_ref[...]
    h = h_ref[...] + t
    hn = _layernorm(h, lnw_ref[...], lnb_ref[...])
    z_ref[...] = jnp.dot(hn, wout_ref[...],
                         preferred_element_type=jnp.float32) + bout_ref[...]


_BLK = 400
_GRID = N // _BLK


def _rowspec(width=F):
    return pl.BlockSpec((_BLK, width), lambda i: (i, 0))


def _fullspec(shape):
    return pl.BlockSpec(shape, lambda i: tuple(0 for _ in shape))


_tc_in = pl.pallas_call(
    _tc_in_body,
    grid=(_GRID,),
    in_specs=[_rowspec(), _fullspec((F, F)), _fullspec((1, F)),
              _fullspec((1, F)), _fullspec((1, F))],
    out_specs=[_rowspec(), _rowspec()],
    out_shape=[jax.ShapeDtypeStruct((N, F), jnp.float32),
               jax.ShapeDtypeStruct((N, F), jnp.float32)],
)

_tc_mid = pl.pallas_call(
    _tc_mid_body,
    grid=(_GRID,),
    in_specs=[pl.BlockSpec((NC, _BLK, F), lambda i: (0, i, 0)),
              _rowspec(), _fullspec((F, F)), _fullspec((1, F)),
              _fullspec((1, F)), _fullspec((1, F))],
    out_specs=[_rowspec(), _rowspec()],
    out_shape=[jax.ShapeDtypeStruct((N, F), jnp.float32),
               jax.ShapeDtypeStruct((N, F), jnp.float32)],
)

_tc_last = pl.pallas_call(
    _tc_last_body,
    grid=(_GRID,),
    in_specs=[pl.BlockSpec((NC, _BLK, F), lambda i: (0, i, 0)),
              _rowspec(), _fullspec((F, F)), _fullspec((1, F)),
              _fullspec((1, F)), _fullspec((1, F)),
              _fullspec((F, NCLASS)), _fullspec((1, NCLASS))],
    out_specs=pl.BlockSpec((_BLK, NCLASS), lambda i: (i, 0)),
    out_shape=jax.ShapeDtypeStruct((N, NCLASS), jnp.float32),
)


def kernel(x, edge_index, edge_weight, only_z, W_in, b_in, W0, b0, lnw0, lnb0,
           W1, b1, lnw1, lnb1, W2, b2, lnw2, lnb2, out_lnw, out_lnb,
           W_out, b_out):
    pad = EPAD - E
    spread = (jnp.arange(pad, dtype=jnp.int32) * 13) % N
    src_p = jnp.concatenate([edge_index[0], spread])
    dst_p = jnp.concatenate([edge_index[1], spread])
    w_p = jnp.concatenate([edge_weight, jnp.zeros((pad,), jnp.float32)])
    srcr = src_p.reshape(NW, NSUP, SCH, C)
    dstr = dst_p.reshape(NW, NSUP, SCH, C)
    wr = w_p.reshape(NW, NSUP, SCH, C)

    r2 = lambda v: v.reshape(1, -1)
    h, hn = _tc_in(x, W_in, r2(b_in), r2(lnw0), r2(lnb0))

    s2 = _spmm(hn, srcr, dstr, wr)
    h, hn = _tc_mid(s2, h, W0, r2(b0), r2(lnw1), r2(lnb1))

    s2 = _spmm(hn, srcr, dstr, wr)
    h, hn = _tc_mid(s2, h, W1, r2(b1), r2(lnw2), r2(lnb2))

    s2 = _spmm(hn, srcr, dstr, wr)
    z = _tc_last(s2, h, W2, r2(b2), r2(out_lnw), r2(out_lnb),
                 W_out, r2(b_out))
    return z * jnp.asarray(only_z, dtype=z.dtype)


# speedup vs baseline: 1.1349x; 1.1349x over previous
import functools

import jax
import jax.numpy as jnp
from jax import lax
from jax.experimental import pallas as pl
from jax.experimental.pallas import tpu as pltpu
from jax.experimental.pallas import tpu_sc as plsc

N = 10000
E = 320000
F = 128
NCLASS = 64

NC = 2
NS = 16
NW = NC * NS
C = 64
EPW = 10368
EPAD = EPW * NW
NCHUNK = EPW // C
SCH = 54
NSUP = NCHUNK // SCH
NBUF = 3
NPAD = 10240
ROWS_PT = NPAD // NS


def _i16(v):
    return jnp.full((16,), v, dtype=jnp.int32)


def _spmm_body(hn, srcr, dstr, wr, out, acc, src_v, dst_v, w_v,
               rows0, rows1, rows2, gsem0, gsem1, gsem2, ssem0, ssem1, ssem2):
    c = lax.axis_index("c")
    s = lax.axis_index("s")
    wid = s * NC + c

    rows = (rows0, rows1, rows2)
    gsems = (gsem0, gsem1, gsem2)
    ssems = (ssem0, ssem1, ssem2)

    def gather_start(k, b):
        pltpu.async_copy(hn.at[src_v.at[k]], rows[b], gsems[b])

    def gather_wait(b):
        pltpu.make_async_copy(hn.at[src_v.at[0]], rows[b], gsems[b]).wait()

    def scatter_start(k, b):
        pltpu.async_copy(rows[b], acc.at[dst_v.at[k]], ssems[b], add=True)

    def scatter_wait(b):
        pltpu.make_async_copy(rows[b], acc.at[dst_v.at[0]], ssems[b]).wait()

    def multiply(k, rows_b):
        def group(g, carry2):
            w16 = w_v[k, pl.ds(g * 16, 16)]
            for j in range(16):
                wvec = jnp.full((16,), w16[j], dtype=jnp.float32)
                e = g * 16 + j
                for f in range(F // 16):
                    sl = (e, pl.ds(f * 16, 16))
                    rows_b[sl] = rows_b[sl] * wvec
            return carry2

        lax.fori_loop(0, C // 16, group, 0, unroll=False)

    def zrow(i, carry):
        for f in range(F // 16):
            rows0[i, pl.ds(f * 16, 16)] = jnp.zeros((16,), jnp.float32)
        return carry

    lax.fori_loop(0, C, zrow, 0, unroll=False)
    for t in range(ROWS_PT // C):
        pltpu.sync_copy(rows0, acc.at[pl.ds(s * ROWS_PT + t * C, C)])
    plsc.subcore_barrier()

    def superchunk(sc_i, carry0):
        pltpu.sync_copy(srcr.at[wid, sc_i], src_v)
        pltpu.sync_copy(dstr.at[wid, sc_i], dst_v)
        pltpu.sync_copy(wr.at[wid, sc_i], w_v)
        gather_start(0, 0)
        gather_start(1, 1)

        def triple(q, carry):
            for b in range(NBUF):
                k = 3 * q + b
                bn = (b + 2) % NBUF
                gather_wait(b)
                multiply(k, rows[b])
                scatter_start(k, b)

                @pl.when(k > 0)
                def _():
                    scatter_wait(bn)

                @pl.when(k + 2 < SCH)
                def _():
                    gather_start(k + 2, bn)
            return carry

        lax.fori_loop(0, SCH // NBUF, triple, 0, unroll=False)
        scatter_wait((SCH - 1) % NBUF)
        return carry0

    lax.fori_loop(0, NSUP, superchunk, 0, unroll=False)
    plsc.subcore_barrier()
    pltpu.sync_copy(acc.at[pl.ds(s * ROWS_PT, ROWS_PT)],
                    out.at[c, pl.ds(s * ROWS_PT, ROWS_PT)])


_spmm = pl.kernel(
    _spmm_body,
    out_type=jax.ShapeDtypeStruct((NC, NPAD, F), jnp.float32),
    mesh=plsc.VectorSubcoreMesh(core_axis_name="c", subcore_axis_name="s"),
    scratch_types=[
        pltpu.VMEM_SHARED((NPAD, F), jnp.float32),
        pltpu.VMEM((SCH, C), jnp.int32),
        pltpu.VMEM((SCH, C), jnp.int32),
        pltpu.VMEM((SCH, C), jnp.float32),
        pltpu.VMEM((C, F), jnp.float32),
        pltpu.VMEM((C, F), jnp.float32),
        pltpu.VMEM((C, F), jnp.float32),
        pltpu.SemaphoreType.DMA,
        pltpu.SemaphoreType.DMA,
        pltpu.SemaphoreType.DMA,
        pltpu.SemaphoreType.DMA,
        pltpu.SemaphoreType.DMA,
        pltpu.SemaphoreType.DMA,
    ],
)


def _layernorm(h, w, b):
    m = jnp.mean(h, axis=-1, keepdims=True)
    v = jnp.mean((h - m) * (h - m), axis=-1, keepdims=True)
    return (h - m) * lax.rsqrt(v + 1e-5) * w + b


def _tc_in_body(x_ref, w_ref, b_ref, lnw_ref, lnb_ref, h_ref, hn_ref):
    h = jnp.dot(x_ref[...], w_ref[...], preferred_element_type=jnp.float32)
    h = jnp.maximum(h + b_ref[...], 0.0)
    h_ref[...] = h
    hn_ref[...] = _layernorm(h, lnw_ref[...], lnb_ref[...])


def _tc_mid_body(s2_ref, h_ref, w_ref, b_ref, lnw_ref, lnb_ref, ho_ref, hn_ref):
    sp = s2_ref[0] + s2_ref[1]
    t = jnp.dot(sp, w_ref[...], preferred_element_type=jnp.float32)
    t = jnp.maximum(t + b_ref[...], 0.0)
    h = h_ref[...] + t
    ho_ref[...] = h
    hn_ref[...] = _layernorm(h, lnw_ref[...], lnb_ref[...])


def _tc_last_body(s2_ref, h_ref, w_ref, b_ref, lnw_ref, lnb_ref,
                  wout_ref, bout_ref, z_ref):
    sp = s2_ref[0] + s2_ref[1]
    t = jnp.dot(sp, w_ref[...], preferred_element_type=jnp.float32) + b_ref[...]
    h = h_ref[...] + t
    hn = _layernorm(h, lnw_ref[...], lnb_ref[...])
    z_ref[...] = jnp.dot(hn, wout_ref[...],
                         preferred_element_type=jnp.float32) + bout_ref[...]


_BLK = 400
_GRID = N // _BLK


def _rowspec(width=F):
    return pl.BlockSpec((_BLK, width), lambda i: (i, 0))


def _fullspec(shape):
    return pl.BlockSpec(shape, lambda i: tuple(0 for _ in shape))


_tc_in = pl.pallas_call(
    _tc_in_body,
    grid=(_GRID,),
    in_specs=[_rowspec(), _fullspec((F, F)), _fullspec((1, F)),
              _fullspec((1, F)), _fullspec((1, F))],
    out_specs=[_rowspec(), _rowspec()],
    out_shape=[jax.ShapeDtypeStruct((N, F), jnp.float32),
               jax.ShapeDtypeStruct((N, F), jnp.float32)],
)

_tc_mid = pl.pallas_call(
    _tc_mid_body,
    grid=(_GRID,),
    in_specs=[pl.BlockSpec((NC, _BLK, F), lambda i: (0, i, 0)),
              _rowspec(), _fullspec((F, F)), _fullspec((1, F)),
              _fullspec((1, F)), _fullspec((1, F))],
    out_specs=[_rowspec(), _rowspec()],
    out_shape=[jax.ShapeDtypeStruct((N, F), jnp.float32),
               jax.ShapeDtypeStruct((N, F), jnp.float32)],
)

_tc_last = pl.pallas_call(
    _tc_last_body,
    grid=(_GRID,),
    in_specs=[pl.BlockSpec((NC, _BLK, F), lambda i: (0, i, 0)),
              _rowspec(), _fullspec((F, F)), _fullspec((1, F)),
              _fullspec((1, F)), _fullspec((1, F)),
              _fullspec((F, NCLASS)), _fullspec((1, NCLASS))],
    out_specs=pl.BlockSpec((_BLK, NCLASS), lambda i: (i, 0)),
    out_shape=jax.ShapeDtypeStruct((N, NCLASS), jnp.float32),
)


def kernel(x, edge_index, edge_weight, only_z, W_in, b_in, W0, b0, lnw0, lnb0,
           W1, b1, lnw1, lnb1, W2, b2, lnw2, lnb2, out_lnw, out_lnb,
           W_out, b_out):
    pad = EPAD - E
    spread = (jnp.arange(pad, dtype=jnp.int32) * 13) % N
    src_p = jnp.concatenate([edge_index[0], spread])
    dst_p = jnp.concatenate([edge_index[1], spread])
    w_p = jnp.concatenate([edge_weight, jnp.zeros((pad,), jnp.float32)])
    srcr = src_p.reshape(NW, NSUP, SCH, C)
    dstr = dst_p.reshape(NW, NSUP, SCH, C)
    wr = w_p.reshape(NW, NSUP, SCH, C)

    r2 = lambda v: v.reshape(1, -1)
    h, hn = _tc_in(x, W_in, r2(b_in), r2(lnw0), r2(lnb0))

    s2 = _spmm(hn, srcr, dstr, wr)
    h, hn = _tc_mid(s2, h, W0, r2(b0), r2(lnw1), r2(lnb1))

    s2 = _spmm(hn, srcr, dstr, wr)
    h, hn = _tc_mid(s2, h, W1, r2(b1), r2(lnw2), r2(lnb2))

    s2 = _spmm(hn, srcr, dstr, wr)
    z = _tc_last(s2, h, W2, r2(b2), r2(out_lnw), r2(out_lnb),
                 W_out, r2(b_out))
    return z * jnp.asarray(only_z, dtype=z.dtype)
